# baseline (speedup 1.0000x reference)
import math

import jax
import jax.numpy as jnp
from jax.experimental import pallas as pl
from jax.experimental.pallas import tpu as pltpu

EPS = 1e-8
B, S, D = 2, 2048, 1024
NH = 4
DH = D // NH
BLK = 4
NQKV = D // BLK
T = 1024
NBLK = S // T
LN_EPS = 1e-5
INV_SQRT_DH = 1.0 / math.sqrt(DH)


def _scan_lanes(v, op, fill):
    r, s = v.shape
    d = 1
    while d < s:
        pad = jnp.full((r, d), fill, dtype=v.dtype)
        v = op(v, jnp.concatenate([pad, v[:, : s - d]], axis=1))
        d *= 2
    return v


def _compute_gates(x_full_ref, wg_ref, bg_ref, gs_s):
    for b in range(B):
        t = jax.lax.dot_general(
            wg_ref[...], x_full_ref[b], (((0,), (1,)), ((), ())),
            preferred_element_type=jnp.float32)
        t = t + bg_ref[...]
        ig = t[0:NH]
        fg = t[NH:2 * NH]
        lf = jnp.minimum(fg, 0.0) - jnp.log1p(jnp.exp(-jnp.abs(fg)))
        cs = _scan_lanes(lf, jnp.add, 0.0)
        m = ig - cs
        mx = _scan_lanes(m, jnp.maximum, -1e30)
        nf = jnp.exp(-(cs + mx))
        rows = jnp.concatenate(
            [jnp.concatenate([m[h:h + 1], mx[h:h + 1], nf[h:h + 1]], axis=0)
             for h in range(NH)], axis=0)
        t2 = jnp.swapaxes(rows, 0, 1)
        for h in range(NH):
            gs_s[b * NH + h, :, :] = t2[:, 3 * h:3 * h + 3]


def _mlstm_kernel(x_ref, x_full_ref, wq_ref, wk_ref, wv_ref, wg_ref, bg_ref,
                  gam_ref, o_ref, ke_s, v_s, gs_s):
    bf16 = jnp.bfloat16
    pid = pl.program_id(0)

    @pl.when(pid == 0)
    def _():
        _compute_gates(x_full_ref, wg_ref, bg_ref, gs_s)

    ii = jax.lax.broadcasted_iota(jnp.int32, (T, T), 0)
    jj = jax.lax.broadcasted_iota(jnp.int32, (T, T), 1)
    causal = ii >= jj

    maxm = []
    for J in range(NBLK):
        sl = slice(J * T, (J + 1) * T)
        mJ = gs_s[pid, sl, 0:1]
        mx = jnp.max(mJ)
        maxm.append(mx)
        e = jnp.exp(mJ - mx)
        xb = x_ref[0, sl, :].astype(bf16)
        kJ = jnp.dot(xb, wk_ref[0], preferred_element_type=jnp.float32)
        ke_s[sl, :] = (kJ * e).astype(bf16)
        v_s[sl, :] = jnp.dot(xb, wv_ref[0],
                             preferred_element_type=jnp.float32
                             ).astype(bf16)

    cmax = [maxm[0]]
    for J in range(1, NBLK):
        cmax.append(jnp.maximum(cmax[J - 1], maxm[J]))

    for I in range(NBLK):
        sli = slice(I * T, (I + 1) * T)
        qI = jnp.dot(x_ref[0, sli, :].astype(bf16), wq_ref[0],
                     preferred_element_type=jnp.float32).astype(bf16)
        MI = gs_s[pid, sli, 1:2]
        nfI = gs_s[pid, sli, 2:3]
        wrow = jnp.exp(jnp.concatenate(
            [jnp.reshape(maxm[J] - cmax[I], (1, 1)) for J in range(I + 1)],
            axis=1))
        acc = jnp.zeros((T, DH), jnp.float32)
        ssum = jnp.zeros((T, 1), jnp.float32)
        for J in range(I + 1):
            slj = slice(J * T, (J + 1) * T)
            s = jax.lax.dot_general(
                qI, ke_s[slj, :], (((1,), (1,)), ((), ())),
                preferred_element_type=jnp.float32)
            if J == I:
                s = jnp.where(causal, s, 0.0)
            w = wrow[0, J]
            ssum = ssum + jnp.sum(s, axis=1, keepdims=True) * w
            c = s.astype(bf16) * w.astype(bf16)
            acc = acc + jnp.dot(c, v_s[slj, :],
                                preferred_element_type=jnp.float32)
        g = jnp.exp(jnp.minimum(cmax[I] - MI, 80.0))
        norm = jnp.maximum(jnp.abs(ssum * g), nfI) + EPS
        hI = acc * (g / norm)
        mu = jnp.mean(hI, axis=1, keepdims=True)
        var = jnp.mean((hI - mu) * (hI - mu), axis=1, keepdims=True)
        hn = (hI - mu) * jax.lax.rsqrt(var + LN_EPS)
        o_ref[0, sli, :] = hn * gam_ref[0]


@jax.jit
def kernel(x, wq, wk, wv, wi, bi, wf, bf, ln_w):
    f32 = jnp.float32
    bf16 = jnp.bfloat16
    nhb = DH // BLK
    wqkv = jnp.concatenate([wq * INV_SQRT_DH, wk, wv], axis=0).astype(f32)
    wt = wqkv.reshape(3 * NH, nhb, BLK, BLK).transpose(0, 1, 3, 2)
    p1 = jnp.pad(wt, ((0, 0), (0, 0), (0, 0), (0, (nhb - 1) * BLK)))
    p2 = p1.reshape(3 * NH, nhb, BLK * DH)
    p3 = jnp.pad(p2, ((0, 0), (0, 0), (0, BLK)))
    p4 = p3.reshape(3 * NH, nhb * (BLK * DH + BLK))[:, :DH * DH]
    w_d = p4.reshape(3 * NH, DH, DH).astype(bf16)
    wi_eff = wi[:, :D] + wi[:, D:2 * D] + wi[:, 2 * D:]
    wf_eff = wf[:, :D] + wf[:, D:2 * D] + wf[:, 2 * D:]
    wg = jnp.concatenate([wi_eff, wf_eff], axis=0).T.astype(f32)
    bg = jnp.concatenate([bi, bf]).reshape(2 * NH, 1).astype(f32)
    gam = (1.0 + ln_w).reshape(NH, 1, DH).astype(f32)

    out = pl.pallas_call(
        _mlstm_kernel,
        out_shape=jax.ShapeDtypeStruct((B, S, D), f32),
        grid=(B * NH,),
        in_specs=[
            pl.BlockSpec((1, S, DH), lambda i: (i // NH, 0, i % NH)),
            pl.BlockSpec((B, S, D), lambda i: (0, 0, 0)),
            pl.BlockSpec((1, DH, DH), lambda i: (i % NH, 0, 0)),
            pl.BlockSpec((1, DH, DH), lambda i: (NH + i % NH, 0, 0)),
            pl.BlockSpec((1, DH, DH), lambda i: (2 * NH + i % NH, 0, 0)),
            pl.BlockSpec((D, 2 * NH), lambda i: (0, 0)),
            pl.BlockSpec((2 * NH, 1), lambda i: (0, 0)),
            pl.BlockSpec((1, 1, DH), lambda i: (i % NH, 0, 0)),
        ],
        out_specs=pl.BlockSpec((1, S, DH), lambda i: (i // NH, 0, i % NH)),
        scratch_shapes=[
            pltpu.VMEM((S, DH), bf16),
            pltpu.VMEM((S, DH), bf16),
            pltpu.VMEM((B * NH, S, 3), f32),
        ],
        compiler_params=pltpu.CompilerParams(
            dimension_semantics=("arbitrary",),
            vmem_limit_bytes=56 * 1024 * 1024,
        ),
        name="mlstm_fused",
    )(x, x, w_d, w_d, w_d, wg, bg, gam)
    return out


# speedup vs baseline: 1.1921x; 1.1921x over previous
import math

import jax
import jax.numpy as jnp
from jax.experimental import pallas as pl
from jax.experimental.pallas import tpu as pltpu

EPS = 1e-8
B, S, D = 2, 2048, 1024
NH = 4
DH = D // NH
BLK = 4
NQKV = D // BLK
T = 512
NBLK = S // T
LN_EPS = 1e-5
INV_SQRT_DH = 1.0 / math.sqrt(DH)


def _scan_lanes(v, op, fill):
    r, s = v.shape
    d = 1
    while d < s:
        pad = jnp.full((r, d), fill, dtype=v.dtype)
        v = op(v, jnp.concatenate([pad, v[:, : s - d]], axis=1))
        d *= 2
    return v


def _compute_gates(x_full_ref, wg_ref, bg_ref, gs_s):
    for b in range(B):
        t = jax.lax.dot_general(
            wg_ref[...], x_full_ref[b], (((0,), (1,)), ((), ())),
            preferred_element_type=jnp.float32)
        t = t + bg_ref[...]
        ig = t[0:NH]
        fg = t[NH:2 * NH]
        lf = jnp.minimum(fg, 0.0) - jnp.log1p(jnp.exp(-jnp.abs(fg)))
        cs = _scan_lanes(lf, jnp.add, 0.0)
        m = ig - cs
        mx = _scan_lanes(m, jnp.maximum, -1e30)
        nf = jnp.exp(-(cs + mx))
        rows = jnp.concatenate(
            [jnp.concatenate([m[h:h + 1], mx[h:h + 1], nf[h:h + 1]], axis=0)
             for h in range(NH)], axis=0)
        t2 = jnp.swapaxes(rows, 0, 1)
        for h in range(NH):
            gs_s[b * NH + h, :, :] = t2[:, 3 * h:3 * h + 3]


def _mlstm_kernel(x_ref, x_full_ref, wq_ref, wk_ref, wv_ref, wg_ref, bg_ref,
                  gam_ref, o_ref, ke_s, v_s, gs_s):
    bf16 = jnp.bfloat16
    pid = pl.program_id(0)

    @pl.when(pid == 0)
    def _():
        _compute_gates(x_full_ref, wg_ref, bg_ref, gs_s)

    ii = jax.lax.broadcasted_iota(jnp.int32, (T, T), 0)
    jj = jax.lax.broadcasted_iota(jnp.int32, (T, T), 1)
    causal = ii >= jj

    maxm = []
    for J in range(NBLK):
        sl = slice(J * T, (J + 1) * T)
        mJ = gs_s[pid, sl, 0:1]
        mx = jnp.max(mJ)
        maxm.append(mx)
        e = jnp.exp(mJ - mx)
        xb = x_ref[0, sl, :].astype(bf16)
        kJ = jnp.dot(xb, wk_ref[0], preferred_element_type=jnp.float32)
        ke_s[sl, :] = (kJ * e).astype(bf16)
        v_s[sl, :] = jnp.dot(xb, wv_ref[0],
                             preferred_element_type=jnp.float32
                             ).astype(bf16)

    cmax = [maxm[0]]
    for J in range(1, NBLK):
        cmax.append(jnp.maximum(cmax[J - 1], maxm[J]))

    for I in range(NBLK):
        sli = slice(I * T, (I + 1) * T)
        qI = jnp.dot(x_ref[0, sli, :].astype(bf16), wq_ref[0],
                     preferred_element_type=jnp.float32).astype(bf16)
        MI = gs_s[pid, sli, 1:2]
        nfI = gs_s[pid, sli, 2:3]
        wrow = jnp.exp(jnp.concatenate(
            [jnp.reshape(maxm[J] - cmax[I], (1, 1)) for J in range(I + 1)],
            axis=1))
        acc = jnp.zeros((T, DH), jnp.float32)
        ssum = jnp.zeros((T, 1), jnp.float32)
        for J in range(I + 1):
            slj = slice(J * T, (J + 1) * T)
            s = jax.lax.dot_general(
                qI, ke_s[slj, :], (((1,), (1,)), ((), ())),
                preferred_element_type=jnp.float32)
            if J == I:
                s = jnp.where(causal, s, 0.0)
            w = wrow[0, J]
            ssum = ssum + jnp.sum(s, axis=1, keepdims=True) * w
            c = s.astype(bf16) * w.astype(bf16)
            acc = acc + jnp.dot(c, v_s[slj, :],
                                preferred_element_type=jnp.float32)
        g = jnp.exp(jnp.minimum(cmax[I] - MI, 80.0))
        norm = jnp.maximum(jnp.abs(ssum * g), nfI) + EPS
        hI = acc * (g / norm)
        mu = jnp.mean(hI, axis=1, keepdims=True)
        var = jnp.mean((hI - mu) * (hI - mu), axis=1, keepdims=True)
        hn = (hI - mu) * jax.lax.rsqrt(var + LN_EPS)
        o_ref[0, sli, :] = hn * gam_ref[0]


@jax.jit
def kernel(x, wq, wk, wv, wi, bi, wf, bf, ln_w):
    f32 = jnp.float32
    bf16 = jnp.bfloat16
    nhb = DH // BLK
    wqkv = jnp.concatenate([wq * INV_SQRT_DH, wk, wv], axis=0).astype(f32)
    wt = wqkv.reshape(3 * NH, nhb, BLK, BLK).transpose(0, 1, 3, 2)
    p1 = jnp.pad(wt, ((0, 0), (0, 0), (0, 0), (0, (nhb - 1) * BLK)))
    p2 = p1.reshape(3 * NH, nhb, BLK * DH)
    p3 = jnp.pad(p2, ((0, 0), (0, 0), (0, BLK)))
    p4 = p3.reshape(3 * NH, nhb * (BLK * DH + BLK))[:, :DH * DH]
    w_d = p4.reshape(3 * NH, DH, DH).astype(bf16)
    wi_eff = wi[:, :D] + wi[:, D:2 * D] + wi[:, 2 * D:]
    wf_eff = wf[:, :D] + wf[:, D:2 * D] + wf[:, 2 * D:]
    wg = jnp.concatenate([wi_eff, wf_eff], axis=0).T.astype(f32)
    bg = jnp.concatenate([bi, bf]).reshape(2 * NH, 1).astype(f32)
    gam = (1.0 + ln_w).reshape(NH, 1, DH).astype(f32)

    out = pl.pallas_call(
        _mlstm_kernel,
        out_shape=jax.ShapeDtypeStruct((B, S, D), f32),
        grid=(B * NH,),
        in_specs=[
            pl.BlockSpec((1, S, DH), lambda i: (i // NH, 0, i % NH)),
            pl.BlockSpec((B, S, D), lambda i: (0, 0, 0)),
            pl.BlockSpec((1, DH, DH), lambda i: (i % NH, 0, 0)),
            pl.BlockSpec((1, DH, DH), lambda i: (NH + i % NH, 0, 0)),
            pl.BlockSpec((1, DH, DH), lambda i: (2 * NH + i % NH, 0, 0)),
            pl.BlockSpec((D, 2 * NH), lambda i: (0, 0)),
            pl.BlockSpec((2 * NH, 1), lambda i: (0, 0)),
            pl.BlockSpec((1, 1, DH), lambda i: (i % NH, 0, 0)),
        ],
        out_specs=pl.BlockSpec((1, S, DH), lambda i: (i // NH, 0, i % NH)),
        scratch_shapes=[
            pltpu.VMEM((S, DH), bf16),
            pltpu.VMEM((S, DH), bf16),
            pltpu.VMEM((B * NH, S, 3), f32),
        ],
        compiler_params=pltpu.CompilerParams(
            dimension_semantics=("arbitrary",),
            vmem_limit_bytes=56 * 1024 * 1024,
        ),
        name="mlstm_fused",
    )(x, x, w_d, w_d, w_d, wg, bg, gam)
    return out


# speedup vs baseline: 1.2527x; 1.0508x over previous
import math

import jax
import jax.numpy as jnp
from jax.experimental import pallas as pl
from jax.experimental.pallas import tpu as pltpu

EPS = 1e-8
B, S, D = 2, 2048, 1024
NH = 4
DH = D // NH
BLK = 4
NQKV = D // BLK
T = 512
NBLK = S // T
LN_EPS = 1e-5
INV_SQRT_DH = 1.0 / math.sqrt(DH)


def _scan_lanes(v, op, fill):
    r, s = v.shape
    d = 1
    while d < s:
        pad = jnp.full((r, d), fill, dtype=v.dtype)
        v = op(v, jnp.concatenate([pad, v[:, : s - d]], axis=1))
        d *= 2
    return v


def _compute_gates(x_full_ref, wi_ref, wf_ref, bg_ref, gs_s):
    wie = wi_ref[:, 0:D] + wi_ref[:, D:2 * D] + wi_ref[:, 2 * D:3 * D]
    wfe = wf_ref[:, 0:D] + wf_ref[:, D:2 * D] + wf_ref[:, 2 * D:3 * D]
    wgk = jnp.concatenate([wie, wfe], axis=0)
    for b in range(B):
        t = jax.lax.dot_general(
            wgk, x_full_ref[b], (((1,), (1,)), ((), ())),
            preferred_element_type=jnp.float32)
        t = t + bg_ref[...]
        ig = t[0:NH]
        fg = t[NH:2 * NH]
        lf = jnp.minimum(fg, 0.0) - jnp.log1p(jnp.exp(-jnp.abs(fg)))
        cs = _scan_lanes(lf, jnp.add, 0.0)
        m = ig - cs
        mx = _scan_lanes(m, jnp.maximum, -1e30)
        nf = jnp.exp(-(cs + mx))
        rows = jnp.concatenate(
            [jnp.concatenate([m[h:h + 1], mx[h:h + 1], nf[h:h + 1]], axis=0)
             for h in range(NH)], axis=0)
        t2 = jnp.swapaxes(rows, 0, 1)
        for h in range(NH):
            gs_s[b * NH + h, :, :] = t2[:, 3 * h:3 * h + 3]


def _mlstm_kernel(x_ref, x_full_ref, wq_ref, wk_ref, wv_ref, wi_ref, wf_ref,
                  bg_ref, lnw_ref, o_ref, ke_s, v_s, gs_s):
    bf16 = jnp.bfloat16
    pid = pl.program_id(0)

    @pl.when(pid == 0)
    def _():
        _compute_gates(x_full_ref, wi_ref, wf_ref, bg_ref, gs_s)

    ii = jax.lax.broadcasted_iota(jnp.int32, (T, T), 0)
    jj = jax.lax.broadcasted_iota(jnp.int32, (T, T), 1)
    causal = ii >= jj

    maxm = []
    for J in range(NBLK):
        sl = slice(J * T, (J + 1) * T)
        mJ = gs_s[pid, sl, 0:1]
        mx = jnp.max(mJ)
        maxm.append(mx)
        e = jnp.exp(mJ - mx)
        xb = x_ref[0, sl, :].astype(bf16)
        kJ = jnp.dot(xb, wk_ref[0], preferred_element_type=jnp.float32)
        ke_s[sl, :] = (kJ * e).astype(bf16)
        v_s[sl, :] = jnp.dot(xb, wv_ref[0],
                             preferred_element_type=jnp.float32
                             ).astype(bf16)

    cmax = [maxm[0]]
    for J in range(1, NBLK):
        cmax.append(jnp.maximum(cmax[J - 1], maxm[J]))

    for I in range(NBLK):
        sli = slice(I * T, (I + 1) * T)
        qI = jnp.dot(x_ref[0, sli, :].astype(bf16), wq_ref[0],
                     preferred_element_type=jnp.float32).astype(bf16)
        MI = gs_s[pid, sli, 1:2]
        nfI = gs_s[pid, sli, 2:3]
        wrow = jnp.exp(jnp.concatenate(
            [jnp.reshape(maxm[J] - cmax[I], (1, 1)) for J in range(I + 1)],
            axis=1))
        acc = jnp.zeros((T, DH), jnp.float32)
        ssum = jnp.zeros((T, 1), jnp.float32)
        for J in range(I + 1):
            slj = slice(J * T, (J + 1) * T)
            s = jax.lax.dot_general(
                qI, ke_s[slj, :], (((1,), (1,)), ((), ())),
                preferred_element_type=jnp.float32)
            if J == I:
                s = jnp.where(causal, s, 0.0)
            w = wrow[0, J]
            ssum = ssum + jnp.sum(s, axis=1, keepdims=True) * w
            c = s.astype(bf16) * w.astype(bf16)
            acc = acc + jnp.dot(c, v_s[slj, :],
                                preferred_element_type=jnp.float32)
        g = jnp.exp(jnp.minimum(cmax[I] - MI, 80.0))
        norm = jnp.maximum(jnp.abs(ssum * g), nfI) + EPS
        hI = acc * (g / norm)
        mu = jnp.mean(hI, axis=1, keepdims=True)
        var = jnp.mean((hI - mu) * (hI - mu), axis=1, keepdims=True)
        hn = (hI - mu) * jax.lax.rsqrt(var + LN_EPS)
        o_ref[0, sli, :] = hn * (1.0 + lnw_ref[0])


@jax.jit
def kernel(x, wq, wk, wv, wi, bi, wf, bf, ln_w):
    f32 = jnp.float32
    bf16 = jnp.bfloat16
    nhb = DH // BLK
    wqkv = jnp.concatenate([wq * INV_SQRT_DH, wk, wv], axis=0).astype(f32)
    wt = wqkv.reshape(3 * NH, nhb, BLK, BLK).transpose(0, 1, 3, 2)
    p1 = jnp.pad(wt, ((0, 0), (0, 0), (0, 0), (0, (nhb - 1) * BLK)))
    p2 = p1.reshape(3 * NH, nhb, BLK * DH)
    p3 = jnp.pad(p2, ((0, 0), (0, 0), (0, BLK)))
    p4 = p3.reshape(3 * NH, nhb * (BLK * DH + BLK))[:, :DH * DH]
    w_d = p4.reshape(3 * NH, DH, DH).astype(bf16)
    bg = jnp.concatenate([bi, bf]).reshape(2 * NH, 1).astype(f32)
    lnw = ln_w.reshape(NH, 1, DH).astype(f32)

    out = pl.pallas_call(
        _mlstm_kernel,
        out_shape=jax.ShapeDtypeStruct((B, S, D), f32),
        grid=(B * NH,),
        in_specs=[
            pl.BlockSpec((1, S, DH), lambda i: (i // NH, 0, i % NH)),
            pl.BlockSpec((B, S, D), lambda i: (0, 0, 0)),
            pl.BlockSpec((1, DH, DH), lambda i: (i % NH, 0, 0)),
            pl.BlockSpec((1, DH, DH), lambda i: (NH + i % NH, 0, 0)),
            pl.BlockSpec((1, DH, DH), lambda i: (2 * NH + i % NH, 0, 0)),
            pl.BlockSpec((NH, 3 * D), lambda i: (0, 0)),
            pl.BlockSpec((NH, 3 * D), lambda i: (0, 0)),
            pl.BlockSpec((2 * NH, 1), lambda i: (0, 0)),
            pl.BlockSpec((1, 1, DH), lambda i: (i % NH, 0, 0)),
        ],
        out_specs=pl.BlockSpec((1, S, DH), lambda i: (i // NH, 0, i % NH)),
        scratch_shapes=[
            pltpu.VMEM((S, DH), bf16),
            pltpu.VMEM((S, DH), bf16),
            pltpu.VMEM((B * NH, S, 3), f32),
        ],
        compiler_params=pltpu.CompilerParams(
            dimension_semantics=("arbitrary",),
            vmem_limit_bytes=56 * 1024 * 1024,
        ),
        name="mlstm_fused",
    )(x, x, w_d, w_d, w_d, wi, wf, bg, lnw)
    return out
